# baseline (speedup 1.0000x reference)
import functools

import jax
import jax.numpy as jnp
from jax import lax
from jax.experimental import pallas as pl
from jax.experimental.pallas import tpu as pltpu
from jax.experimental.pallas import tpu_sc as plsc

_SHORT = 65536
_MED = 16384
_LONG = 4096
_DIM = 512
_TOTAL = _SHORT + _MED + _LONG
_B = 8192

_L = 16
_NC = 2
_NS = 16
_NW = _NC * _NS

_SLICE = _SHORT // _NW
_NCHUNK = _B // _L
_WCHUNK = _SLICE // _L

_LOG_B = 13
_LOG_SLICE = 11

_BLK = 256
_N_SHORT = _SHORT // _BLK
_N_MED = _MED // _BLK
_N_LONG = _LONG // _BLK
_N_TOT = _TOTAL // _BLK


_HD = 8
_ND = 2 * _HD


def _in_copy(j, slot, short_ref, med_ref, long_ref, buf, in_sems):
    @pl.when(j < _N_SHORT)
    def _():
        pltpu.make_async_copy(
            short_ref.at[pl.ds(j * _BLK, _BLK)], buf.at[slot],
            in_sems.at[slot]).start()

    @pl.when(jnp.logical_and(j >= _N_SHORT, j < _N_SHORT + _N_MED))
    def _():
        pltpu.make_async_copy(
            med_ref.at[pl.ds((j - _N_SHORT) * _BLK, _BLK)], buf.at[slot],
            in_sems.at[slot]).start()

    @pl.when(j >= _N_SHORT + _N_MED)
    def _():
        pltpu.make_async_copy(
            long_ref.at[pl.ds((j - _N_SHORT - _N_MED) * _BLK, _BLK)],
            buf.at[slot], in_sems.at[slot]).start()


def _assemble_body(short_ref, med_ref, long_ref, out_ref, buf, in_sems, out_sems):
    def loop_body(j, carry):
        @pl.when(j < _N_TOT)
        def _():
            slot = lax.rem(j, _ND)

            @pl.when(j >= _ND)
            def _():
                pltpu.make_async_copy(
                    buf.at[slot], out_ref.at[pl.ds(0, _BLK)],
                    out_sems.at[slot]).wait()

            _in_copy(j, slot, short_ref, med_ref, long_ref, buf, in_sems)

        jo = j - _HD

        @pl.when(jnp.logical_and(jo >= 0, jo < _N_TOT))
        def _():
            slot = lax.rem(jo, _ND)
            pltpu.make_async_copy(
                buf.at[slot], out_ref.at[pl.ds(0, _BLK)],
                in_sems.at[slot]).wait()
            pltpu.make_async_copy(
                buf.at[slot], out_ref.at[pl.ds(jo * _BLK, _BLK)],
                out_sems.at[slot]).start()
        return carry

    lax.fori_loop(0, _N_TOT + _HD, loop_body, 0, unroll=False)
    for k in range(_ND):
        pltpu.make_async_copy(
            buf.at[k], out_ref.at[pl.ds(0, _BLK)], out_sems.at[k]).wait()


def _assemble(short_mem, medium_mem, long_mem):
    return pl.pallas_call(
        _assemble_body,
        in_specs=[
            pl.BlockSpec(memory_space=pl.ANY),
            pl.BlockSpec(memory_space=pl.ANY),
            pl.BlockSpec(memory_space=pl.ANY),
        ],
        out_specs=pl.BlockSpec(memory_space=pl.ANY),
        out_shape=jax.ShapeDtypeStruct((_TOTAL, _DIM), jnp.float32),
        scratch_shapes=[
            pltpu.VMEM((_ND, _BLK, _DIM), jnp.float32),
            pltpu.SemaphoreType.DMA((_ND,)),
            pltpu.SemaphoreType.DMA((_ND,)),
        ],
    )(short_mem, medium_mem, long_mem)


def _sc_plan_body(idx_hbm, cl_hbm, cnt_hbm,
                  idx_v, w_v, cl_v, shift_v, cnt_v, sem_idx, sem_out):
    cid = lax.axis_index("c")
    sid = lax.axis_index("s")
    wid = sid * _NC + cid

    idx_cp = pltpu.make_async_copy(idx_hbm, idx_v, sem_idx)
    idx_cp.start()
    idx_cp.wait()

    iota = lax.iota(jnp.int32, _L)
    minus1 = jnp.full((_L,), -1, jnp.int32)

    def init_body(v, carry):
        w_v[pl.ds(v * _L, _L)] = minus1
        return carry

    lax.fori_loop(0, _WCHUNK, init_body, 0, unroll=False)

    def scan_body(c, carry):
        iv = idx_v[pl.ds(c * _L, _L)]
        shift_v[...] = iv
        drop = iota < 0
        for sh in range(1, _L):
            nb = plsc.load_gather(shift_v, [jnp.minimum(iota + sh, _L - 1)])
            drop = drop | ((iv == nb) & (iota + sh < _L))
        mine = lax.shift_right_logical(iv, _LOG_SLICE) == wid
        keep = mine & jnp.logical_not(drop)
        addr = jnp.where(keep, iv & (_SLICE - 1), _SLICE + iota)
        plsc.store_scatter(w_v, [addr], iota + c * _L)
        return carry

    lax.fori_loop(0, _NCHUNK, scan_body, 0, unroll=False)

    def compact_body(v, base):
        wv = w_v[pl.ds(v * _L, _L)]
        m = wv >= 0
        cnt = plsc.all_reduce_population_count(m)
        rank = plsc.cumsum(m.astype(jnp.int32)) - 1
        gdest = wid * _SLICE + v * _L + iota
        comb = gdest * _B + jnp.where(m, wv, 0)
        addr = jnp.where(m, base + rank, _SLICE + iota)
        plsc.store_scatter(cl_v, [addr], comb, mask=m)
        return base + cnt

    base = lax.fori_loop(0, _WCHUNK, compact_body,
                         jnp.zeros((_L,), jnp.int32), unroll=False)

    cnt_v[...] = base
    cl_cp = pltpu.make_async_copy(
        cl_v.at[pl.ds(0, _SLICE)], cl_hbm.at[wid], sem_out)
    cl_cp.start()
    cnt_cp = pltpu.make_async_copy(cnt_v, cnt_hbm.at[wid], sem_out)
    cnt_cp.start()
    cl_cp.wait()
    cnt_cp.wait()


_CD = 8


def _sc_emit_body(upd_hbm, cl_hbm, cnt_hbm, out_hbm,
                  cl_v, cnt_v, row_buf, sem_in, gsems, ssems):
    cid = lax.axis_index("c")
    sid = lax.axis_index("s")
    wid = sid * _NC + cid

    cl_cp = pltpu.make_async_copy(cl_hbm.at[wid], cl_v, sem_in)
    cl_cp.start()
    cnt_cp = pltpu.make_async_copy(cnt_hbm.at[wid], cnt_v, sem_in)
    cnt_cp.start()
    cl_cp.wait()
    cnt_cp.wait()

    n = jnp.max(cnt_v[...])
    nc = lax.shift_right_logical(n + (_L - 1), 4)
    iota = lax.iota(jnp.int32, _L)

    def emit_body(c, dest_prev):
        active = c < nc
        cidx = jnp.minimum(c, _WCHUNK - 1)
        cl = cl_v[pl.ds(cidx * _L, _L)]
        valid = (cidx * _L + iota) < n
        cm = jnp.max(jnp.where(valid, cl, -1))
        clf = jnp.where(valid, cl, cm)
        dest = lax.shift_right_logical(clf, _LOG_B)
        fp = clf & (_B - 1)

        @pl.when(active)
        def _():
            slot = lax.rem(c, _CD)

            @pl.when(c >= _CD)
            def _():
                pltpu.make_async_copy(
                    row_buf.at[slot], out_hbm.at[pl.ds(0, _L)],
                    ssems.at[slot]).wait()

            pltpu.make_async_copy(
                upd_hbm.at[fp], row_buf.at[slot], gsems.at[slot]).start()

        cp = c - 1

        @pl.when(jnp.logical_and(cp >= 0, cp < nc))
        def _():
            slot = lax.rem(cp, _CD)
            pltpu.make_async_copy(
                upd_hbm.at[pl.ds(0, _L)], row_buf.at[slot],
                gsems.at[slot]).wait()
            pltpu.make_async_copy(
                row_buf.at[slot], out_hbm.at[dest_prev], ssems.at[slot]).start()

        return jnp.where(active, dest, dest_prev)

    lax.fori_loop(0, _WCHUNK + 1, emit_body,
                  jnp.zeros((_L,), jnp.int32), unroll=False)

    for k in range(_CD):
        @pl.when(k < nc)
        def _():
            pltpu.make_async_copy(
                row_buf.at[k], out_hbm.at[pl.ds(0, _L)],
                ssems.at[lax.rem(nc - 1 - k, _CD)]).wait()


def _make_sc_kernels():
    mesh = plsc.VectorSubcoreMesh(core_axis_name="c", subcore_axis_name="s")
    plan = pl.kernel(
        _sc_plan_body,
        out_type=(
            jax.ShapeDtypeStruct((_NW, _SLICE), jnp.int32),
            jax.ShapeDtypeStruct((_NW, _L), jnp.int32),
        ),
        mesh=mesh,
        scratch_types=[
            pltpu.VMEM((_B,), jnp.int32),
            pltpu.VMEM((_SLICE + _L,), jnp.int32),
            pltpu.VMEM((_SLICE + _L,), jnp.int32),
            pltpu.VMEM((_L,), jnp.int32),
            pltpu.VMEM((_L,), jnp.int32),
            pltpu.SemaphoreType.DMA,
            pltpu.SemaphoreType.DMA,
        ],
        compiler_params=pltpu.CompilerParams(needs_layout_passes=False),
        name="sc_scatter_plan",
    )
    emit = pl.kernel(
        _sc_emit_body,
        out_type=(),
        mesh=mesh,
        scratch_types=[
            pltpu.VMEM((_SLICE,), jnp.int32),
            pltpu.VMEM((_L,), jnp.int32),
            pltpu.VMEM((_CD, _L, _DIM), jnp.float32),
            pltpu.SemaphoreType.DMA,
            pltpu.SemaphoreType.DMA((_CD,)),
            pltpu.SemaphoreType.DMA((_CD,)),
        ],
        compiler_params=pltpu.CompilerParams(needs_layout_passes=False),
        name="sc_scatter_emit",
    )
    return plan, emit


_sc_plan, _sc_emit = _make_sc_kernels()


def kernel(updates, short_idx, short_mem, medium_mem, long_mem):
    idx32 = short_idx.astype(jnp.int32)
    cl, cnt = _sc_plan(idx32)
    assembled = _assemble(short_mem, medium_mem, long_mem)
    out_ref = jax.new_ref(assembled)
    _sc_emit(updates, cl, cnt, out_ref)
    return jax.freeze(out_ref)


# speedup vs baseline: 1.0126x; 1.0126x over previous
import functools

import jax
import jax.numpy as jnp
from jax import lax
from jax.experimental import pallas as pl
from jax.experimental.pallas import tpu as pltpu
from jax.experimental.pallas import tpu_sc as plsc

_SHORT = 65536
_MED = 16384
_LONG = 4096
_DIM = 512
_TOTAL = _SHORT + _MED + _LONG
_B = 8192

_L = 16
_NC = 2
_NS = 16
_NW = _NC * _NS

_SLICE = _SHORT // _NW
_NCHUNK = _B // _L
_WCHUNK = _SLICE // _L

_LOG_B = 13
_LOG_SLICE = 11

_BLK = 512
_N_SHORT = _SHORT // _BLK
_N_MED = _MED // _BLK
_N_LONG = _LONG // _BLK
_N_TOT = _TOTAL // _BLK


_HD = 8
_ND = 2 * _HD


def _in_copy(j, slot, short_ref, med_ref, long_ref, buf, in_sems):
    @pl.when(j < _N_SHORT)
    def _():
        pltpu.make_async_copy(
            short_ref.at[pl.ds(j * _BLK, _BLK)], buf.at[slot],
            in_sems.at[slot]).start()

    @pl.when(jnp.logical_and(j >= _N_SHORT, j < _N_SHORT + _N_MED))
    def _():
        pltpu.make_async_copy(
            med_ref.at[pl.ds((j - _N_SHORT) * _BLK, _BLK)], buf.at[slot],
            in_sems.at[slot]).start()

    @pl.when(j >= _N_SHORT + _N_MED)
    def _():
        pltpu.make_async_copy(
            long_ref.at[pl.ds((j - _N_SHORT - _N_MED) * _BLK, _BLK)],
            buf.at[slot], in_sems.at[slot]).start()


def _assemble_body(short_ref, med_ref, long_ref, out_ref, buf, in_sems, out_sems):
    def loop_body(j, carry):
        @pl.when(j < _N_TOT)
        def _():
            slot = lax.rem(j, _ND)

            @pl.when(j >= _ND)
            def _():
                pltpu.make_async_copy(
                    buf.at[slot], out_ref.at[pl.ds(0, _BLK)],
                    out_sems.at[slot]).wait()

            _in_copy(j, slot, short_ref, med_ref, long_ref, buf, in_sems)

        jo = j - _HD

        @pl.when(jnp.logical_and(jo >= 0, jo < _N_TOT))
        def _():
            slot = lax.rem(jo, _ND)
            pltpu.make_async_copy(
                buf.at[slot], out_ref.at[pl.ds(0, _BLK)],
                in_sems.at[slot]).wait()
            pltpu.make_async_copy(
                buf.at[slot], out_ref.at[pl.ds(jo * _BLK, _BLK)],
                out_sems.at[slot]).start()
        return carry

    lax.fori_loop(0, _N_TOT + _HD, loop_body, 0, unroll=False)
    for k in range(_ND):
        pltpu.make_async_copy(
            buf.at[k], out_ref.at[pl.ds(0, _BLK)], out_sems.at[k]).wait()


def _assemble(short_mem, medium_mem, long_mem):
    return pl.pallas_call(
        _assemble_body,
        in_specs=[
            pl.BlockSpec(memory_space=pl.ANY),
            pl.BlockSpec(memory_space=pl.ANY),
            pl.BlockSpec(memory_space=pl.ANY),
        ],
        out_specs=pl.BlockSpec(memory_space=pl.ANY),
        out_shape=jax.ShapeDtypeStruct((_TOTAL, _DIM), jnp.float32),
        scratch_shapes=[
            pltpu.VMEM((_ND, _BLK, _DIM), jnp.float32),
            pltpu.SemaphoreType.DMA((_ND,)),
            pltpu.SemaphoreType.DMA((_ND,)),
        ],
    )(short_mem, medium_mem, long_mem)


def _sc_plan_body(idx_hbm, cl_hbm, cnt_hbm,
                  idx_v, w_v, cl_v, shift_v, cnt_v, sem_idx, sem_out):
    cid = lax.axis_index("c")
    sid = lax.axis_index("s")
    wid = sid * _NC + cid

    idx_cp = pltpu.make_async_copy(idx_hbm, idx_v, sem_idx)
    idx_cp.start()
    idx_cp.wait()

    iota = lax.iota(jnp.int32, _L)
    minus1 = jnp.full((_L,), -1, jnp.int32)

    def init_body(v, carry):
        w_v[pl.ds(v * _L, _L)] = minus1
        return carry

    lax.fori_loop(0, _WCHUNK, init_body, 0, unroll=False)

    def scan_body(c, carry):
        iv = idx_v[pl.ds(c * _L, _L)]
        shift_v[...] = iv
        drop = iota < 0
        for sh in range(1, _L):
            nb = plsc.load_gather(shift_v, [jnp.minimum(iota + sh, _L - 1)])
            drop = drop | ((iv == nb) & (iota + sh < _L))
        mine = lax.shift_right_logical(iv, _LOG_SLICE) == wid
        keep = mine & jnp.logical_not(drop)
        addr = jnp.where(keep, iv & (_SLICE - 1), _SLICE + iota)
        plsc.store_scatter(w_v, [addr], iota + c * _L)
        return carry

    lax.fori_loop(0, _NCHUNK, scan_body, 0, unroll=False)

    def compact_body(v, base):
        wv = w_v[pl.ds(v * _L, _L)]
        m = wv >= 0
        cnt = plsc.all_reduce_population_count(m)
        rank = plsc.cumsum(m.astype(jnp.int32)) - 1
        gdest = wid * _SLICE + v * _L + iota
        comb = gdest * _B + jnp.where(m, wv, 0)
        addr = jnp.where(m, base + rank, _SLICE + iota)
        plsc.store_scatter(cl_v, [addr], comb, mask=m)
        return base + cnt

    base = lax.fori_loop(0, _WCHUNK, compact_body,
                         jnp.zeros((_L,), jnp.int32), unroll=False)

    cnt_v[...] = base
    cl_cp = pltpu.make_async_copy(
        cl_v.at[pl.ds(0, _SLICE)], cl_hbm.at[wid], sem_out)
    cl_cp.start()
    cnt_cp = pltpu.make_async_copy(cnt_v, cnt_hbm.at[wid], sem_out)
    cnt_cp.start()
    cl_cp.wait()
    cnt_cp.wait()


_CD = 8


def _sc_emit_body(upd_hbm, cl_hbm, cnt_hbm, out_hbm,
                  cl_v, cnt_v, row_buf, sem_in, gsems, ssems):
    cid = lax.axis_index("c")
    sid = lax.axis_index("s")
    wid = sid * _NC + cid

    cl_cp = pltpu.make_async_copy(cl_hbm.at[wid], cl_v, sem_in)
    cl_cp.start()
    cnt_cp = pltpu.make_async_copy(cnt_hbm.at[wid], cnt_v, sem_in)
    cnt_cp.start()
    cl_cp.wait()
    cnt_cp.wait()

    n = jnp.max(cnt_v[...])
    nc = lax.shift_right_logical(n + (_L - 1), 4)
    iota = lax.iota(jnp.int32, _L)

    def emit_body(c, dest_prev):
        active = c < nc
        cidx = jnp.minimum(c, _WCHUNK - 1)
        cl = cl_v[pl.ds(cidx * _L, _L)]
        valid = (cidx * _L + iota) < n
        cm = jnp.max(jnp.where(valid, cl, -1))
        clf = jnp.where(valid, cl, cm)
        dest = lax.shift_right_logical(clf, _LOG_B)
        fp = clf & (_B - 1)

        @pl.when(active)
        def _():
            slot = lax.rem(c, _CD)

            @pl.when(c >= _CD)
            def _():
                pltpu.make_async_copy(
                    row_buf.at[slot], out_hbm.at[pl.ds(0, _L)],
                    ssems.at[slot]).wait()

            pltpu.make_async_copy(
                upd_hbm.at[fp], row_buf.at[slot], gsems.at[slot]).start()

        cp = c - 1

        @pl.when(jnp.logical_and(cp >= 0, cp < nc))
        def _():
            slot = lax.rem(cp, _CD)
            pltpu.make_async_copy(
                upd_hbm.at[pl.ds(0, _L)], row_buf.at[slot],
                gsems.at[slot]).wait()
            pltpu.make_async_copy(
                row_buf.at[slot], out_hbm.at[dest_prev], ssems.at[slot]).start()

        return jnp.where(active, dest, dest_prev)

    lax.fori_loop(0, _WCHUNK + 1, emit_body,
                  jnp.zeros((_L,), jnp.int32), unroll=False)

    for k in range(_CD):
        @pl.when(k < nc)
        def _():
            pltpu.make_async_copy(
                row_buf.at[k], out_hbm.at[pl.ds(0, _L)],
                ssems.at[lax.rem(nc - 1 - k, _CD)]).wait()


def _make_sc_kernels():
    mesh = plsc.VectorSubcoreMesh(core_axis_name="c", subcore_axis_name="s")
    plan = pl.kernel(
        _sc_plan_body,
        out_type=(
            jax.ShapeDtypeStruct((_NW, _SLICE), jnp.int32),
            jax.ShapeDtypeStruct((_NW, _L), jnp.int32),
        ),
        mesh=mesh,
        scratch_types=[
            pltpu.VMEM((_B,), jnp.int32),
            pltpu.VMEM((_SLICE + _L,), jnp.int32),
            pltpu.VMEM((_SLICE + _L,), jnp.int32),
            pltpu.VMEM((_L,), jnp.int32),
            pltpu.VMEM((_L,), jnp.int32),
            pltpu.SemaphoreType.DMA,
            pltpu.SemaphoreType.DMA,
        ],
        compiler_params=pltpu.CompilerParams(needs_layout_passes=False),
        name="sc_scatter_plan",
    )
    emit = pl.kernel(
        _sc_emit_body,
        out_type=(),
        mesh=mesh,
        scratch_types=[
            pltpu.VMEM((_SLICE,), jnp.int32),
            pltpu.VMEM((_L,), jnp.int32),
            pltpu.VMEM((_CD, _L, _DIM), jnp.float32),
            pltpu.SemaphoreType.DMA,
            pltpu.SemaphoreType.DMA((_CD,)),
            pltpu.SemaphoreType.DMA((_CD,)),
        ],
        compiler_params=pltpu.CompilerParams(needs_layout_passes=False),
        name="sc_scatter_emit",
    )
    return plan, emit


_sc_plan, _sc_emit = _make_sc_kernels()


def kernel(updates, short_idx, short_mem, medium_mem, long_mem):
    idx32 = short_idx.astype(jnp.int32)
    cl, cnt = _sc_plan(idx32)
    assembled = _assemble(short_mem, medium_mem, long_mem)
    out_ref = jax.new_ref(assembled)
    _sc_emit(updates, cl, cnt, out_ref)
    return jax.freeze(out_ref)


# speedup vs baseline: 1.0132x; 1.0005x over previous
import jax
import jax.numpy as jnp
from jax import lax
from jax.experimental import pallas as pl
from jax.experimental.pallas import tpu as pltpu
from jax.experimental.pallas import tpu_sc as plsc

_SHORT = 65536
_MED = 16384
_LONG = 4096
_DIM = 512
_TOTAL = _SHORT + _MED + _LONG
_B = 8192

_L = 16
_NC = 2
_NS = 16
_NW = _NC * _NS

_SLICE = _SHORT // _NW
_NCHUNK = _B // _L
_WCHUNK = _SLICE // _L

_LOG_B = 13
_LOG_SLICE = 11

_BLK = 512
_N_SHORT = _SHORT // _BLK
_N_MED = _MED // _BLK
_N_LONG = _LONG // _BLK
_N_TOT = _TOTAL // _BLK


_HD = 8
_ND = 2 * _HD


def _in_copy(j, slot, short_ref, med_ref, long_ref, buf, in_sems):
    @pl.when(j < _N_SHORT)
    def _():
        pltpu.make_async_copy(
            short_ref.at[pl.ds(j * _BLK, _BLK)], buf.at[slot],
            in_sems.at[slot]).start()

    @pl.when(jnp.logical_and(j >= _N_SHORT, j < _N_SHORT + _N_MED))
    def _():
        pltpu.make_async_copy(
            med_ref.at[pl.ds((j - _N_SHORT) * _BLK, _BLK)], buf.at[slot],
            in_sems.at[slot]).start()

    @pl.when(j >= _N_SHORT + _N_MED)
    def _():
        pltpu.make_async_copy(
            long_ref.at[pl.ds((j - _N_SHORT - _N_MED) * _BLK, _BLK)],
            buf.at[slot], in_sems.at[slot]).start()


def _assemble_body(short_ref, med_ref, long_ref, out_ref, buf, in_sems, out_sems):
    def loop_body(j, carry):
        @pl.when(j < _N_TOT)
        def _():
            slot = lax.rem(j, _ND)

            @pl.when(j >= _ND)
            def _():
                pltpu.make_async_copy(
                    buf.at[slot], out_ref.at[pl.ds(0, _BLK)],
                    out_sems.at[slot]).wait()

            _in_copy(j, slot, short_ref, med_ref, long_ref, buf, in_sems)

        jo = j - _HD

        @pl.when(jnp.logical_and(jo >= 0, jo < _N_TOT))
        def _():
            slot = lax.rem(jo, _ND)
            pltpu.make_async_copy(
                buf.at[slot], out_ref.at[pl.ds(0, _BLK)],
                in_sems.at[slot]).wait()
            pltpu.make_async_copy(
                buf.at[slot], out_ref.at[pl.ds(jo * _BLK, _BLK)],
                out_sems.at[slot]).start()
        return carry

    lax.fori_loop(0, _N_TOT + _HD, loop_body, 0, unroll=False)
    for k in range(_ND):
        pltpu.make_async_copy(
            buf.at[k], out_ref.at[pl.ds(0, _BLK)], out_sems.at[k]).wait()


def _assemble(short_mem, medium_mem, long_mem):
    return pl.pallas_call(
        _assemble_body,
        in_specs=[
            pl.BlockSpec(memory_space=pl.ANY),
            pl.BlockSpec(memory_space=pl.ANY),
            pl.BlockSpec(memory_space=pl.ANY),
        ],
        out_specs=pl.BlockSpec(memory_space=pl.ANY),
        out_shape=jax.ShapeDtypeStruct((_TOTAL, _DIM), jnp.float32),
        scratch_shapes=[
            pltpu.VMEM((_ND, _BLK, _DIM), jnp.float32),
            pltpu.SemaphoreType.DMA((_ND,)),
            pltpu.SemaphoreType.DMA((_ND,)),
        ],
    )(short_mem, medium_mem, long_mem)


def _sc_plan_body(idx_hbm, cl_hbm, cnt_hbm,
                  idx_v, w_v, cl_v, shift_v, cnt_v, sem_idx, sem_out):
    cid = lax.axis_index("c")
    sid = lax.axis_index("s")
    wid = sid * _NC + cid

    idx_cp = pltpu.make_async_copy(idx_hbm, idx_v, sem_idx)
    idx_cp.start()
    idx_cp.wait()

    iota = lax.iota(jnp.int32, _L)
    minus1 = jnp.full((_L,), -1, jnp.int32)

    def init_body(v, carry):
        w_v[pl.ds(v * _L, _L)] = minus1
        return carry

    lax.fori_loop(0, _WCHUNK, init_body, 0, unroll=False)

    def scan_body(c, carry):
        iv = idx_v[pl.ds(c * _L, _L)]
        shift_v[...] = iv
        drop = iota < 0
        for sh in range(1, _L):
            nb = plsc.load_gather(shift_v, [jnp.minimum(iota + sh, _L - 1)])
            drop = drop | ((iv == nb) & (iota + sh < _L))
        mine = lax.shift_right_logical(iv, _LOG_SLICE) == wid
        keep = mine & jnp.logical_not(drop)
        addr = jnp.where(keep, iv & (_SLICE - 1), _SLICE + iota)
        plsc.store_scatter(w_v, [addr], iota + c * _L)
        return carry

    lax.fori_loop(0, _NCHUNK, scan_body, 0, unroll=False)

    def compact_body(v, base):
        wv = w_v[pl.ds(v * _L, _L)]
        m = wv >= 0
        cnt = plsc.all_reduce_population_count(m)
        rank = plsc.cumsum(m.astype(jnp.int32)) - 1
        gdest = wid * _SLICE + v * _L + iota
        comb = gdest * _B + jnp.where(m, wv, 0)
        addr = jnp.where(m, base + rank, _SLICE + iota)
        plsc.store_scatter(cl_v, [addr], comb, mask=m)
        return base + cnt

    base = lax.fori_loop(0, _WCHUNK, compact_body,
                         jnp.zeros((_L,), jnp.int32), unroll=False)

    cnt_v[...] = base
    cl_cp = pltpu.make_async_copy(
        cl_v.at[pl.ds(0, _SLICE)], cl_hbm.at[wid], sem_out)
    cl_cp.start()
    cnt_cp = pltpu.make_async_copy(cnt_v, cnt_hbm.at[wid], sem_out)
    cnt_cp.start()
    cl_cp.wait()
    cnt_cp.wait()


_CD = 8


def _sc_emit_body(upd_hbm, cl_hbm, cnt_hbm, out_hbm,
                  cl_v, cnt_v, row_buf, sem_in, gsems, ssems):
    cid = lax.axis_index("c")
    sid = lax.axis_index("s")
    wid = sid * _NC + cid

    cl_cp = pltpu.make_async_copy(cl_hbm.at[wid], cl_v, sem_in)
    cl_cp.start()
    cnt_cp = pltpu.make_async_copy(cnt_hbm.at[wid], cnt_v, sem_in)
    cnt_cp.start()
    cl_cp.wait()
    cnt_cp.wait()

    n = jnp.max(cnt_v[...])
    nc = lax.shift_right_logical(n + (_L - 1), 4)
    iota = lax.iota(jnp.int32, _L)

    def emit_body(c, dest_prev):
        active = c < nc
        cidx = jnp.minimum(c, _WCHUNK - 1)
        cl = cl_v[pl.ds(cidx * _L, _L)]
        valid = (cidx * _L + iota) < n
        cm = jnp.max(jnp.where(valid, cl, -1))
        clf = jnp.where(valid, cl, cm)
        dest = lax.shift_right_logical(clf, _LOG_B)
        fp = clf & (_B - 1)

        @pl.when(active)
        def _():
            slot = lax.rem(c, _CD)

            @pl.when(c >= _CD)
            def _():
                pltpu.make_async_copy(
                    row_buf.at[slot], out_hbm.at[pl.ds(0, _L)],
                    ssems.at[slot]).wait()

            pltpu.make_async_copy(
                upd_hbm.at[fp], row_buf.at[slot], gsems.at[slot]).start()

        cp = c - 1

        @pl.when(jnp.logical_and(cp >= 0, cp < nc))
        def _():
            slot = lax.rem(cp, _CD)
            pltpu.make_async_copy(
                upd_hbm.at[pl.ds(0, _L)], row_buf.at[slot],
                gsems.at[slot]).wait()
            pltpu.make_async_copy(
                row_buf.at[slot], out_hbm.at[dest_prev], ssems.at[slot]).start()

        return jnp.where(active, dest, dest_prev)

    lax.fori_loop(0, _WCHUNK + 1, emit_body,
                  jnp.zeros((_L,), jnp.int32), unroll=False)

    for k in range(_CD):
        @pl.when(k < nc)
        def _():
            pltpu.make_async_copy(
                row_buf.at[k], out_hbm.at[pl.ds(0, _L)],
                ssems.at[lax.rem(nc - 1 - k, _CD)]).wait()


def _make_sc_kernels():
    mesh = plsc.VectorSubcoreMesh(core_axis_name="c", subcore_axis_name="s")
    plan = pl.kernel(
        _sc_plan_body,
        out_type=(
            jax.ShapeDtypeStruct((_NW, _SLICE), jnp.int32),
            jax.ShapeDtypeStruct((_NW, _L), jnp.int32),
        ),
        mesh=mesh,
        scratch_types=[
            pltpu.VMEM((_B,), jnp.int32),
            pltpu.VMEM((_SLICE + _L,), jnp.int32),
            pltpu.VMEM((_SLICE + _L,), jnp.int32),
            pltpu.VMEM((_L,), jnp.int32),
            pltpu.VMEM((_L,), jnp.int32),
            pltpu.SemaphoreType.DMA,
            pltpu.SemaphoreType.DMA,
        ],
        compiler_params=pltpu.CompilerParams(needs_layout_passes=False),
        name="sc_scatter_plan",
    )
    emit = pl.kernel(
        _sc_emit_body,
        out_type=(),
        mesh=mesh,
        scratch_types=[
            pltpu.VMEM((_SLICE,), jnp.int32),
            pltpu.VMEM((_L,), jnp.int32),
            pltpu.VMEM((_CD, _L, _DIM), jnp.float32),
            pltpu.SemaphoreType.DMA,
            pltpu.SemaphoreType.DMA((_CD,)),
            pltpu.SemaphoreType.DMA((_CD,)),
        ],
        compiler_params=pltpu.CompilerParams(needs_layout_passes=False),
        name="sc_scatter_emit",
    )
    return plan, emit


_sc_plan, _sc_emit = _make_sc_kernels()


def kernel(updates, short_idx, short_mem, medium_mem, long_mem):
    idx32 = short_idx.astype(jnp.int32)
    cl, cnt = _sc_plan(idx32)
    assembled = _assemble(short_mem, medium_mem, long_mem)
    out_ref = jax.new_ref(assembled)
    _sc_emit(updates, cl, cnt, out_ref)
    return jax.freeze(out_ref)
